# baseline (speedup 1.0000x reference)
import functools

import jax
import jax.numpy as jnp
from jax.experimental import pallas as pl
from jax.experimental.pallas import tpu as pltpu
from jax.experimental.pallas import tpu_sc as plsc

B = 16384
HID = 1024
HYP = 128
M = 16
BB = 256
BM = BB * M
W = 128


def _tc_body(hid_ref, hyp_ref, Wp_ref, bp_ref, gamma_ref, beta_ref,
             W1_ref, b1_ref, W2_ref, b2_ref, Wgh_ref, Wgn_ref, bg_ref,
             upd_ref, conf_ref, min_ref, max_ref, g_ref, v_ref):
    hid = hid_ref[...]
    X = hyp_ref[...]

    nh = jnp.dot(hid, Wp_ref[...], preferred_element_type=jnp.float32) + bp_ref[...]
    mu = jnp.mean(nh, axis=-1, keepdims=True)
    var = jnp.mean((nh - mu) ** 2, axis=-1, keepdims=True)
    nh = (nh - mu) * jax.lax.rsqrt(var + 1e-5) * gamma_ref[...] + beta_ref[...]

    h1 = jnp.maximum(
        jnp.dot(X, W1_ref[...], preferred_element_type=jnp.float32) + b1_ref[...],
        0.0)
    logit = jnp.sum((h1 * W2_ref[...]).reshape(BB, M, HYP // 2), axis=2)
    conf = jax.nn.sigmoid(logit + b2_ref[...])
    conf_ref[...] = conf

    iota = jax.lax.broadcasted_iota(jnp.int32, (BB, M), 1)
    cmin = jnp.min(conf, axis=1, keepdims=True)
    cmax = jnp.max(conf, axis=1, keepdims=True)
    min_ref[...] = jnp.min(jnp.where(conf == cmin, iota, M), axis=1, keepdims=True)
    max_ref[...] = jnp.min(jnp.where(conf == cmax, iota, M), axis=1, keepdims=True)

    g = jax.nn.sigmoid(
        jnp.dot(hid, Wgh_ref[...], preferred_element_type=jnp.float32)
        + jnp.dot(nh, Wgn_ref[...], preferred_element_type=jnp.float32)
        + bg_ref[...])
    g_ref[...] = g
    v_ref[...] = (1.0 - g) * nh

    upd_ref[...] = X


_VECTOR_MESH = plsc.VectorSubcoreMesh(core_axis_name="c", subcore_axis_name="s")


@functools.partial(
    pl.kernel,
    out_type=jax.ShapeDtypeStruct((B, HYP), jnp.float32),
    mesh=_VECTOR_MESH,
    scratch_types=[pltpu.VMEM((W, HYP), jnp.float32)],
)
def _sc_update(upd_hbm, rmin_hbm, rmax_hbm, g_hbm, v_hbm, prim_hbm, old_v):
    def body(rmin_v, rmax_v, g_v, v_v, prim_v):
        pltpu.sync_copy(upd_hbm.at[rmin_v.at[0]], old_v)
        def blend_row(i, carry):
            for j in range(HYP // 16):
                sl = pl.ds(j * 16, 16)
                old_v[i, sl] = g_v[i, sl] * old_v[i, sl] + v_v[i, sl]
            return carry
        jax.lax.fori_loop(0, W, blend_row, 0)
        pltpu.sync_copy(old_v, upd_hbm.at[rmin_v.at[0]])
        pltpu.sync_copy(upd_hbm.at[rmax_v.at[0]], prim_v)

    pltpu.emit_pipeline(
        body,
        grid=(B // W,),
        in_specs=[
            pl.BlockSpec((1, W), lambda i: (0, i)),
            pl.BlockSpec((1, W), lambda i: (0, i)),
            pl.BlockSpec((W, HYP), lambda i: (i, 0)),
            pl.BlockSpec((W, HYP), lambda i: (i, 0)),
        ],
        out_specs=[pl.BlockSpec((W, HYP), lambda i: (i, 0))],
        core_axis_name="s",
        dimension_semantics=(pltpu.PARALLEL,),
    )(rmin_hbm, rmax_hbm, g_hbm, v_hbm, prim_hbm)


@jax.jit
def _run(hidden_state, hypotheses, Wp, bp, gamma, beta, W1, b1, W2, b2, Wg, bg):
    Wgh = Wg[:HID]
    Wgn = Wg[HID:]
    bp2 = bp.reshape(1, HYP)
    gamma2 = gamma.reshape(1, HYP)
    beta2 = beta.reshape(1, HYP)
    b12 = b1.reshape(1, HYP // 2)
    W22 = W2.reshape(1, HYP // 2)
    b22 = b2.reshape(1, 1)
    bg2 = bg.reshape(1, HYP)
    hyp2 = hypotheses.reshape(B * M, HYP)

    grid = (B // BB,)
    full = lambda *shape: pl.BlockSpec(shape, lambda i: (0,) * len(shape))
    upd_base, conf, min_idx, max_idx, g, v = pl.pallas_call(
        _tc_body,
        grid=grid,
        in_specs=[
            pl.BlockSpec((BB, HID), lambda i: (i, 0)),
            pl.BlockSpec((BM, HYP), lambda i: (i, 0)),
            full(HID, HYP),
            full(1, HYP),
            full(1, HYP),
            full(1, HYP),
            full(HYP, HYP // 2),
            full(1, HYP // 2),
            full(1, HYP // 2),
            full(1, 1),
            full(HID, HYP),
            full(HYP, HYP),
            full(1, HYP),
        ],
        out_specs=[
            pl.BlockSpec((BM, HYP), lambda i: (i, 0)),
            pl.BlockSpec((BB, M), lambda i: (i, 0)),
            pl.BlockSpec((BB, 1), lambda i: (i, 0)),
            pl.BlockSpec((BB, 1), lambda i: (i, 0)),
            pl.BlockSpec((BB, HYP), lambda i: (i, 0)),
            pl.BlockSpec((BB, HYP), lambda i: (i, 0)),
        ],
        out_shape=[
            jax.ShapeDtypeStruct((B * M, HYP), jnp.float32),
            jax.ShapeDtypeStruct((B, M), jnp.float32),
            jax.ShapeDtypeStruct((B, 1), jnp.int32),
            jax.ShapeDtypeStruct((B, 1), jnp.int32),
            jax.ShapeDtypeStruct((B, HYP), jnp.float32),
            jax.ShapeDtypeStruct((B, HYP), jnp.float32),
        ],
    )(hidden_state, hyp2, Wp, bp2, gamma2, beta2,
      W1, b12, W22, b22, Wgh, Wgn, bg2)

    rows = jnp.arange(B, dtype=jnp.int32) * M
    rmin = (rows + min_idx[:, 0]).reshape(1, B)
    rmax = (rows + max_idx[:, 0]).reshape(1, B)

    upd_ref = jax.new_ref(upd_base)
    primary = _sc_update(upd_ref, rmin, rmax, g, v)
    updated = upd_ref[...].reshape(B, M, HYP)
    return updated, primary, conf


def kernel(hidden_state, hypotheses, Wp, bp, gamma, beta, W1, b1, W2, b2, Wg, bg):
    return _run(hidden_state, hypotheses, Wp, bp, gamma, beta,
                W1, b1, W2, b2, Wg, bg)


# speedup vs baseline: 1.3135x; 1.3135x over previous
import functools

import jax
import jax.numpy as jnp
from jax.experimental import pallas as pl

B = 16384
HID = 1024
HYP = 128
M = 16
BB = 512
BM = BB * M


def _body(iota3_ref, hid_ref, hyp_ref, Wp_ref, bp_ref, gamma_ref, beta_ref,
          W1_ref, b1_ref, W2_ref, b2_ref, Wgh_ref, Wgn_ref, bg_ref,
          upd_ref, prim_ref, conf_ref):
    hid = hid_ref[...]
    X = hyp_ref[...]

    nh = jnp.dot(hid, Wp_ref[...], preferred_element_type=jnp.float32) + bp_ref[...]
    mu = jnp.mean(nh, axis=-1, keepdims=True)
    var = jnp.mean((nh - mu) ** 2, axis=-1, keepdims=True)
    nh = (nh - mu) * jax.lax.rsqrt(var + 1e-5) * gamma_ref[...] + beta_ref[...]

    h1 = jnp.maximum(
        jnp.dot(X, W1_ref[...], preferred_element_type=jnp.float32) + b1_ref[...],
        0.0)
    logit = jnp.sum((h1 * W2_ref[...]).reshape(BB, M, HYP // 2), axis=2)
    conf = jax.nn.sigmoid(logit + b2_ref[...])
    conf_ref[...] = conf

    iota = jax.lax.broadcasted_iota(jnp.int32, (BB, M), 1)
    cmin = jnp.min(conf, axis=1, keepdims=True)
    cmax = jnp.max(conf, axis=1, keepdims=True)
    min_idx = jnp.min(jnp.where(conf == cmin, iota, M), axis=1, keepdims=True)
    max_idx = jnp.min(jnp.where(conf == cmax, iota, M), axis=1, keepdims=True)

    iota3 = iota3_ref[...]
    min3 = jax.lax.broadcast_in_dim(min_idx, (BB, M, HYP), (0, 2))
    max3 = jax.lax.broadcast_in_dim(max_idx, (BB, M, HYP), (0, 2))
    sel_min = iota3 == min3
    sel_max = iota3 == max3

    g = jax.nn.sigmoid(
        jnp.dot(hid, Wgh_ref[...], preferred_element_type=jnp.float32)
        + jnp.dot(nh, Wgn_ref[...], preferred_element_type=jnp.float32)
        + bg_ref[...])
    v = (1.0 - g) * nh

    g_exp = jax.lax.broadcast_in_dim(g, (BB, M, HYP), (0, 2))
    v_exp = jax.lax.broadcast_in_dim(v, (BB, M, HYP), (0, 2))

    X3 = X.reshape(BB, M, HYP)
    upd3 = jnp.where(sel_min, v_exp + g_exp * X3, X3)
    upd_ref[...] = upd3.reshape(BM, HYP)

    prim_ref[...] = jnp.sum(jnp.where(sel_max, upd3, 0.0), axis=1)


@functools.partial(jax.jit, static_argnames=("interpret",))
def _run(hidden_state, hypotheses, Wp, bp, gamma, beta, W1, b1, W2, b2, Wg, bg,
         interpret=False):
    Wgh = Wg[:HID]
    Wgn = Wg[HID:]
    bp2 = bp.reshape(1, HYP)
    gamma2 = gamma.reshape(1, HYP)
    beta2 = beta.reshape(1, HYP)
    b12 = b1.reshape(1, HYP // 2)
    W22 = W2.reshape(1, HYP // 2)
    b22 = b2.reshape(1, 1)
    bg2 = bg.reshape(1, HYP)
    hyp2 = hypotheses.reshape(B * M, HYP)
    iota3 = jax.lax.broadcasted_iota(jnp.int32, (BB, M, HYP), 1)

    grid = (B // BB,)
    full = lambda *shape: pl.BlockSpec(shape, lambda i: (0,) * len(shape))
    out = pl.pallas_call(
        _body,
        grid=grid,
        in_specs=[
            full(BB, M, HYP),
            pl.BlockSpec((BB, HID), lambda i: (i, 0)),
            pl.BlockSpec((BM, HYP), lambda i: (i, 0)),
            full(HID, HYP),
            full(1, HYP),
            full(1, HYP),
            full(1, HYP),
            full(HYP, HYP // 2),
            full(1, HYP // 2),
            full(1, HYP // 2),
            full(1, 1),
            full(HID, HYP),
            full(HYP, HYP),
            full(1, HYP),
        ],
        out_specs=[
            pl.BlockSpec((BM, HYP), lambda i: (i, 0)),
            pl.BlockSpec((BB, HYP), lambda i: (i, 0)),
            pl.BlockSpec((BB, M), lambda i: (i, 0)),
        ],
        out_shape=[
            jax.ShapeDtypeStruct((B * M, HYP), jnp.float32),
            jax.ShapeDtypeStruct((B, HYP), jnp.float32),
            jax.ShapeDtypeStruct((B, M), jnp.float32),
        ],
        interpret=interpret,
    )(iota3, hidden_state, hyp2, Wp, bp2, gamma2, beta2,
      W1, b12, W22, b22, Wgh, Wgn, bg2)
    updated2, primary, conf = out
    return updated2.reshape(B, M, HYP), primary, conf


def kernel(hidden_state, hypotheses, Wp, bp, gamma, beta, W1, b1, W2, b2, Wg, bg):
    return _run(hidden_state, hypotheses, Wp, bp, gamma, beta,
                W1, b1, W2, b2, Wg, bg)


# speedup vs baseline: 1.3545x; 1.0312x over previous
import functools

import jax
import jax.numpy as jnp
from jax.experimental import pallas as pl

B = 16384
HID = 1024
HYP = 128
M = 16
BB = 512
BM = BB * M


def _body(hid_ref, hyp_ref, Wp_ref, bp_ref, gamma_ref, beta_ref,
          W1_ref, b1_ref, W2_ref, b2_ref, Wgh_ref, Wgn_ref, bg_ref,
          upd_ref, prim_ref, conf_ref):
    hid = hid_ref[...]
    X = hyp_ref[...]

    nh = jnp.dot(hid, Wp_ref[...], preferred_element_type=jnp.float32) + bp_ref[...]
    mu = jnp.mean(nh, axis=-1, keepdims=True)
    var = jnp.mean((nh - mu) ** 2, axis=-1, keepdims=True)
    nh = (nh - mu) * jax.lax.rsqrt(var + 1e-5) * gamma_ref[...] + beta_ref[...]

    h1 = jnp.maximum(
        jnp.dot(X, W1_ref[...], preferred_element_type=jnp.float32) + b1_ref[...],
        0.0)
    logit = jnp.sum((h1 * W2_ref[...]).reshape(BB, M, HYP // 2), axis=2)
    conf = jax.nn.sigmoid(logit + b2_ref[...])
    conf_ref[...] = conf

    iota = jax.lax.broadcasted_iota(jnp.int32, (BB, M), 1)
    cmin = jnp.min(conf, axis=1, keepdims=True)
    cmax = jnp.max(conf, axis=1, keepdims=True)
    min_idx = jnp.min(jnp.where(conf == cmin, iota, M), axis=1, keepdims=True)
    max_idx = jnp.min(jnp.where(conf == cmax, iota, M), axis=1, keepdims=True)

    iota3 = jax.lax.broadcasted_iota(jnp.int32, (BB, M, HYP), 1)
    min3 = jax.lax.broadcast_in_dim(min_idx, (BB, M, HYP), (0, 2))
    max3 = jax.lax.broadcast_in_dim(max_idx, (BB, M, HYP), (0, 2))
    sel_min = iota3 == min3
    sel_max = iota3 == max3

    g = jax.nn.sigmoid(
        jnp.dot(hid, Wgh_ref[...], preferred_element_type=jnp.float32)
        + jnp.dot(nh, Wgn_ref[...], preferred_element_type=jnp.float32)
        + bg_ref[...])
    v = (1.0 - g) * nh

    g_exp = jax.lax.broadcast_in_dim(g, (BB, M, HYP), (0, 2))
    v_exp = jax.lax.broadcast_in_dim(v, (BB, M, HYP), (0, 2))

    X3 = X.reshape(BB, M, HYP)
    upd3 = jnp.where(sel_min, v_exp + g_exp * X3, X3)
    upd_ref[...] = upd3.reshape(BM, HYP)

    prim_ref[...] = jnp.sum(jnp.where(sel_max, upd3, 0.0), axis=1)


@functools.partial(jax.jit, static_argnames=("interpret",))
def _run(hidden_state, hypotheses, Wp, bp, gamma, beta, W1, b1, W2, b2, Wg, bg,
         interpret=False):
    Wgh = Wg[:HID]
    Wgn = Wg[HID:]
    bp2 = bp.reshape(1, HYP)
    gamma2 = gamma.reshape(1, HYP)
    beta2 = beta.reshape(1, HYP)
    b12 = b1.reshape(1, HYP // 2)
    W22 = W2.reshape(1, HYP // 2)
    b22 = b2.reshape(1, 1)
    bg2 = bg.reshape(1, HYP)
    hyp2 = hypotheses.reshape(B * M, HYP)

    grid = (B // BB,)
    full = lambda *shape: pl.BlockSpec(shape, lambda i: (0,) * len(shape))
    out = pl.pallas_call(
        _body,
        grid=grid,
        in_specs=[
            pl.BlockSpec((BB, HID), lambda i: (i, 0)),
            pl.BlockSpec((BM, HYP), lambda i: (i, 0)),
            full(HID, HYP),
            full(1, HYP),
            full(1, HYP),
            full(1, HYP),
            full(HYP, HYP // 2),
            full(1, HYP // 2),
            full(1, HYP // 2),
            full(1, 1),
            full(HID, HYP),
            full(HYP, HYP),
            full(1, HYP),
        ],
        out_specs=[
            pl.BlockSpec((BM, HYP), lambda i: (i, 0)),
            pl.BlockSpec((BB, HYP), lambda i: (i, 0)),
            pl.BlockSpec((BB, M), lambda i: (i, 0)),
        ],
        out_shape=[
            jax.ShapeDtypeStruct((B * M, HYP), jnp.float32),
            jax.ShapeDtypeStruct((B, HYP), jnp.float32),
            jax.ShapeDtypeStruct((B, M), jnp.float32),
        ],
        interpret=interpret,
    )(hidden_state, hyp2, Wp, bp2, gamma2, beta2,
      W1, b12, W22, b22, Wgh, Wgn, bg2)
    updated2, primary, conf = out
    return updated2.reshape(B, M, HYP), primary, conf


def kernel(hidden_state, hypotheses, Wp, bp, gamma, beta, W1, b1, W2, b2, Wg, bg):
    return _run(hidden_state, hypotheses, Wp, bp, gamma, beta,
                W1, b1, W2, b2, Wg, bg)


# speedup vs baseline: 1.3562x; 1.0012x over previous
import jax
import jax.numpy as jnp
from jax.experimental import pallas as pl

B = 16384
HID = 1024
HYP = 128
M = 16
BB = 512
BM = BB * M


def _body(hid_ref, hyp_ref, Wp_ref, bp_ref, gamma_ref, beta_ref,
          W1_ref, b1_ref, W2_ref, b2_ref, Wgh_ref, Wgn_ref, bg_ref,
          upd_ref, prim_ref, conf_ref):
    hid = hid_ref[...]
    X = hyp_ref[...]

    nh = jnp.dot(hid, Wp_ref[...], preferred_element_type=jnp.float32) + bp_ref[...]
    mu = jnp.mean(nh, axis=-1, keepdims=True)
    var = jnp.mean((nh - mu) ** 2, axis=-1, keepdims=True)
    nh = (nh - mu) * jax.lax.rsqrt(var + 1e-5) * gamma_ref[...] + beta_ref[...]

    h1 = jnp.maximum(
        jnp.dot(X, W1_ref[...], preferred_element_type=jnp.float32) + b1_ref[...],
        0.0)
    logit = jnp.sum((h1 * W2_ref[...]).reshape(BB, M, HYP // 2), axis=2)
    conf = jax.nn.sigmoid(logit + b2_ref[...])
    conf_ref[...] = conf

    iota = jax.lax.broadcasted_iota(jnp.int32, (BB, M), 1)
    cmin = jnp.min(conf, axis=1, keepdims=True)
    cmax = jnp.max(conf, axis=1, keepdims=True)
    min_idx = jnp.min(jnp.where(conf == cmin, iota, M), axis=1, keepdims=True)
    max_idx = jnp.min(jnp.where(conf == cmax, iota, M), axis=1, keepdims=True)

    iota3 = jax.lax.broadcasted_iota(jnp.int32, (BB, M, HYP), 1)
    min3 = jax.lax.broadcast_in_dim(min_idx, (BB, M, HYP), (0, 2))
    max3 = jax.lax.broadcast_in_dim(max_idx, (BB, M, HYP), (0, 2))
    sel_min = iota3 == min3
    sel_max = iota3 == max3

    g = jax.nn.sigmoid(
        jnp.dot(hid, Wgh_ref[...], preferred_element_type=jnp.float32)
        + jnp.dot(nh, Wgn_ref[...], preferred_element_type=jnp.float32)
        + bg_ref[...])
    v = (1.0 - g) * nh

    g_exp = jax.lax.broadcast_in_dim(g, (BB, M, HYP), (0, 2))
    v_exp = jax.lax.broadcast_in_dim(v, (BB, M, HYP), (0, 2))

    X3 = X.reshape(BB, M, HYP)
    upd3 = jnp.where(sel_min, v_exp + g_exp * X3, X3)
    upd_ref[...] = upd3.reshape(BM, HYP)

    prim_ref[...] = jnp.sum(jnp.where(sel_max, upd3, 0.0), axis=1)


@jax.jit
def _run(hidden_state, hypotheses, Wp, bp, gamma, beta, W1, b1, W2, b2, Wg, bg):
    Wgh = Wg[:HID]
    Wgn = Wg[HID:]
    bp2 = bp.reshape(1, HYP)
    gamma2 = gamma.reshape(1, HYP)
    beta2 = beta.reshape(1, HYP)
    b12 = b1.reshape(1, HYP // 2)
    W22 = W2.reshape(1, HYP // 2)
    b22 = b2.reshape(1, 1)
    bg2 = bg.reshape(1, HYP)
    hyp2 = hypotheses.reshape(B * M, HYP)

    grid = (B // BB,)
    full = lambda *shape: pl.BlockSpec(shape, lambda i: (0,) * len(shape))
    out = pl.pallas_call(
        _body,
        grid=grid,
        in_specs=[
            pl.BlockSpec((BB, HID), lambda i: (i, 0)),
            pl.BlockSpec((BM, HYP), lambda i: (i, 0)),
            full(HID, HYP),
            full(1, HYP),
            full(1, HYP),
            full(1, HYP),
            full(HYP, HYP // 2),
            full(1, HYP // 2),
            full(1, HYP // 2),
            full(1, 1),
            full(HID, HYP),
            full(HYP, HYP),
            full(1, HYP),
        ],
        out_specs=[
            pl.BlockSpec((BM, HYP), lambda i: (i, 0)),
            pl.BlockSpec((BB, HYP), lambda i: (i, 0)),
            pl.BlockSpec((BB, M), lambda i: (i, 0)),
        ],
        out_shape=[
            jax.ShapeDtypeStruct((B * M, HYP), jnp.float32),
            jax.ShapeDtypeStruct((B, HYP), jnp.float32),
            jax.ShapeDtypeStruct((B, M), jnp.float32),
        ],
    )(hidden_state, hyp2, Wp, bp2, gamma2, beta2,
      W1, b12, W22, b22, Wgh, Wgn, bg2)
    updated2, primary, conf = out
    return updated2.reshape(B, M, HYP), primary, conf


def kernel(hidden_state, hypotheses, Wp, bp, gamma, beta, W1, b1, W2, b2, Wg, bg):
    return _run(hidden_state, hypotheses, Wp, bp, gamma, beta,
                W1, b1, W2, b2, Wg, bg)
